# baseline (speedup 1.0000x reference)
import functools

import jax
import jax.numpy as jnp
from jax import lax
from jax.experimental import pallas as pl
from jax.experimental.pallas import tpu as pltpu
from jax.experimental.pallas import tpu_sc as plsc

_LANES = 128
_NS = 16
_CH = 12800


def _round_up(a, b):
    return (a + b - 1) // b * b


def _make_edge_pass(n_pad, e_pad, do_gather):
    rounds = e_pad // (_NS * _CH)
    ts = n_pad // _NS
    mesh = plsc.VectorSubcoreMesh(core_axis_name="c", subcore_axis_name="s",
                                  num_cores=1)

    out_type = jax.ShapeDtypeStruct((n_pad,), jnp.float32)

    if do_gather:
        scratch = [
            pltpu.VMEM((_CH,), jnp.int32),
            pltpu.VMEM((_CH,), jnp.int32),
            pltpu.VMEM((_CH,), jnp.float32),
            pltpu.VMEM_SHARED((n_pad,), jnp.float32),
            pltpu.VMEM_SHARED((n_pad,), jnp.float32),
            pltpu.SemaphoreType.DMA,
        ]

        @functools.partial(pl.kernel, mesh=mesh, out_type=out_type,
                           scratch_types=scratch)
        def k(src_hbm, dst_hbm, table_hbm, zeros_hbm, out_hbm,
              src_v, dst_v, vals_v, acc_sh, tab_sh, sem):
            s = lax.axis_index("s").astype(jnp.int32)
            soff = pl.multiple_of(s * ts, 8)
            pltpu.sync_copy(table_hbm.at[pl.ds(soff, ts)],
                            tab_sh.at[pl.ds(soff, ts)])
            pltpu.sync_copy(zeros_hbm.at[pl.ds(soff, ts)],
                            acc_sh.at[pl.ds(soff, ts)])
            plsc.subcore_barrier()
            tbase = s * jnp.int32(_CH)

            def body(i, carry):
                eoff = pl.multiple_of(tbase + i * jnp.int32(_NS * _CH), 8)
                pltpu.sync_copy(src_hbm.at[pl.ds(eoff, _CH)], src_v)
                pltpu.sync_copy(dst_hbm.at[pl.ds(eoff, _CH)], dst_v)
                pltpu.async_copy(tab_sh.at[src_v], vals_v, sem).wait()
                pltpu.sync_copy(vals_v, acc_sh.at[dst_v], add=True)
                return carry

            lax.fori_loop(jnp.int32(0), jnp.int32(rounds), body, jnp.int32(0))
            plsc.subcore_barrier()
            pltpu.sync_copy(acc_sh.at[pl.ds(soff, ts)],
                            out_hbm.at[pl.ds(soff, ts)])
        return k
    else:
        scratch = [
            pltpu.VMEM((_CH,), jnp.int32),
            pltpu.VMEM((_CH,), jnp.float32),
            pltpu.VMEM_SHARED((n_pad,), jnp.float32),
            pltpu.SemaphoreType.DMA,
        ]

        @functools.partial(pl.kernel, mesh=mesh, out_type=out_type,
                           scratch_types=scratch)
        def k(dst_hbm, ones_hbm, zeros_hbm, out_hbm,
              dst_v, vals_v, acc_sh, sem):
            s = lax.axis_index("s").astype(jnp.int32)
            soff = pl.multiple_of(s * ts, 8)
            pltpu.sync_copy(zeros_hbm.at[pl.ds(soff, ts)],
                            acc_sh.at[pl.ds(soff, ts)])
            pltpu.sync_copy(ones_hbm, vals_v)
            plsc.subcore_barrier()
            tbase = s * jnp.int32(_CH)

            def body(i, carry):
                eoff = pl.multiple_of(tbase + i * jnp.int32(_NS * _CH), 8)
                pltpu.sync_copy(dst_hbm.at[pl.ds(eoff, _CH)], dst_v)
                pltpu.sync_copy(vals_v, acc_sh.at[dst_v], add=True)
                return carry

            lax.fori_loop(jnp.int32(0), jnp.int32(rounds), body, jnp.int32(0))
            plsc.subcore_barrier()
            pltpu.sync_copy(acc_sh.at[pl.ds(soff, ts)],
                            out_hbm.at[pl.ds(soff, ts)])
        return k


def _tc_stage1(deg0, x2):
    rn = x2.shape[0]

    def body(deg_ref, x_ref, dinv_ref, u_ref, invdeg_ref):
        deg = deg_ref[...] + 1.0
        dinv = lax.rsqrt(deg)
        dinv_ref[...] = dinv
        invdeg_ref[...] = 1.0 / deg
        u_ref[...] = dinv * x_ref[...]

    shp = jax.ShapeDtypeStruct((rn, _LANES), jnp.float32)
    return pl.pallas_call(body, out_shape=[shp, shp, shp])(deg0, x2)


def _tc_stage2(acc, dinv, x2, invdeg, aux):
    rn = x2.shape[0]

    def body(acc_ref, dinv_ref, x_ref, invdeg_ref, aux_ref, g_ref, v_ref):
        dinv = dinv_ref[...]
        s = dinv * acc_ref[...] + x_ref[...] * invdeg_ref[...]
        g = jnp.zeros_like(s)
        for j in range(16):
            w1j = aux_ref[0, j]
            b1j = aux_ref[1, j]
            w2j = aux_ref[2, j]
            g = g + jnp.maximum(s * w1j + b1j, 0.0) * w2j
        g_ref[...] = g
        v_ref[...] = dinv * g

    shp = jax.ShapeDtypeStruct((rn, _LANES), jnp.float32)
    return pl.pallas_call(body, out_shape=[shp, shp])(
        acc, dinv, x2, invdeg, aux)


def _tc_stage3(acc, dinv, g, invdeg, aux):
    rn = g.shape[0]

    def body(acc_ref, dinv_ref, g_ref, invdeg_ref, aux_ref, out_ref):
        h = (dinv_ref[...] * acc_ref[...]
             + g_ref[...] * invdeg_ref[...] + aux_ref[3, 0])
        z = h - h
        lse = jnp.log(jnp.exp(z))
        out_ref[...] = z - lse

    shp = jax.ShapeDtypeStruct((rn, _LANES), jnp.float32)
    return pl.pallas_call(body, out_shape=shp)(acc, dinv, g, invdeg, aux)


def kernel(x, edge_index, W1, b1, W2, b2):
    n = x.shape[0]
    e = edge_index.shape[1]
    n_pad = _round_up(n + 1, _LANES * 8)
    rn = n_pad // _LANES
    e_pad = _round_up(e, _NS * _CH)
    dummy = n_pad - 1

    ei = edge_index.astype(jnp.int32)
    pad = jnp.full((e_pad - e,), dummy, jnp.int32)
    src1 = jnp.concatenate([ei[0], pad])
    dst1 = jnp.concatenate([ei[1], pad])

    x1 = jnp.zeros((n_pad,), jnp.float32).at[:n].set(x[:, 0].astype(jnp.float32))
    x2 = x1.reshape(rn, _LANES)
    zeros_n = jnp.zeros((n_pad,), jnp.float32)
    ones_ch = jnp.ones((_CH,), jnp.float32)

    aux = jnp.zeros((8, _LANES), jnp.float32)
    aux = aux.at[0, :16].set(W1[0].astype(jnp.float32))
    aux = aux.at[1, :16].set(b1.astype(jnp.float32))
    aux = aux.at[2, :16].set(W2[:, 0].astype(jnp.float32))
    aux = aux.at[3, 0].set(b2[0].astype(jnp.float32))

    count_k = _make_edge_pass(n_pad, e_pad, do_gather=False)
    gs_k = _make_edge_pass(n_pad, e_pad, do_gather=True)

    deg0 = count_k(dst1, ones_ch, zeros_n)
    dinv, u, invdeg = _tc_stage1(deg0.reshape(rn, _LANES), x2)

    acc1 = gs_k(src1, dst1, u.reshape(n_pad), zeros_n)
    g, v = _tc_stage2(acc1.reshape(rn, _LANES), dinv, x2, invdeg, aux)

    acc2 = gs_k(src1, dst1, v.reshape(n_pad), zeros_n)
    lsm = _tc_stage3(acc2.reshape(rn, _LANES), dinv, g, invdeg, aux)

    return lsm.reshape(n_pad)[:n].reshape(n, 1).astype(jnp.float64)


# speedup vs baseline: 1.8773x; 1.8773x over previous
import functools

import jax
import jax.numpy as jnp
from jax import lax
from jax.experimental import pallas as pl
from jax.experimental.pallas import tpu as pltpu
from jax.experimental.pallas import tpu_sc as plsc

_LANES = 128
_NS = 16
_CH = 10000
_R0_NUM = 11
_R0_DEN = 16


def _round_up(a, b):
    return (a + b - 1) // b * b


def _make_edge_pass(n_pad, e_pad, do_gather):
    rounds = e_pad // (_NS * _CH)
    r0 = rounds * _R0_NUM // _R0_DEN
    r1 = rounds - r0
    ts = n_pad // _NS
    mesh = plsc.VectorSubcoreMesh(core_axis_name="c", subcore_axis_name="s")

    out_type = jax.ShapeDtypeStruct((2, n_pad), jnp.float32)

    if do_gather:
        scratch = [
            pltpu.VMEM((_CH,), jnp.int32),
            pltpu.VMEM((_CH,), jnp.int32),
            pltpu.VMEM((_CH,), jnp.float32),
            pltpu.VMEM_SHARED((n_pad,), jnp.float32),
            pltpu.VMEM_SHARED((n_pad,), jnp.float32),
            pltpu.SemaphoreType.DMA,
        ]

        @functools.partial(pl.kernel, mesh=mesh, out_type=out_type,
                           scratch_types=scratch)
        def k(src_hbm, dst_hbm, table_hbm, zeros_hbm, out_hbm,
              src_v, dst_v, vals_v, acc_sh, tab_sh, sem):
            c = lax.axis_index("c").astype(jnp.int32)
            s = lax.axis_index("s").astype(jnp.int32)
            soff = pl.multiple_of(s * ts, 8)
            pltpu.sync_copy(table_hbm.at[pl.ds(soff, ts)],
                            tab_sh.at[pl.ds(soff, ts)])
            pltpu.sync_copy(zeros_hbm.at[pl.ds(soff, ts)],
                            acc_sh.at[pl.ds(soff, ts)])
            plsc.subcore_barrier()
            my_rounds = jnp.where(c == 0, jnp.int32(r0), jnp.int32(r1))
            cbase = jnp.where(c == 0, jnp.int32(0),
                              jnp.int32(r0)) * jnp.int32(_NS * _CH)
            tbase = cbase + s * jnp.int32(_CH)

            def body(i, carry):
                eoff = pl.multiple_of(tbase + i * jnp.int32(_NS * _CH), 8)
                pltpu.sync_copy(src_hbm.at[pl.ds(eoff, _CH)], src_v)
                pltpu.sync_copy(dst_hbm.at[pl.ds(eoff, _CH)], dst_v)
                pltpu.async_copy(tab_sh.at[src_v], vals_v, sem).wait()
                pltpu.sync_copy(vals_v, acc_sh.at[dst_v], add=True)
                return carry

            lax.fori_loop(jnp.int32(0), my_rounds, body, jnp.int32(0))
            plsc.subcore_barrier()
            pltpu.sync_copy(acc_sh.at[pl.ds(soff, ts)],
                            out_hbm.at[c, pl.ds(soff, ts)])
        return k
    else:
        scratch = [
            pltpu.VMEM((_CH,), jnp.int32),
            pltpu.VMEM((_CH,), jnp.float32),
            pltpu.VMEM_SHARED((n_pad,), jnp.float32),
            pltpu.SemaphoreType.DMA,
        ]

        @functools.partial(pl.kernel, mesh=mesh, out_type=out_type,
                           scratch_types=scratch)
        def k(dst_hbm, ones_hbm, zeros_hbm, out_hbm,
              dst_v, vals_v, acc_sh, sem):
            c = lax.axis_index("c").astype(jnp.int32)
            s = lax.axis_index("s").astype(jnp.int32)
            soff = pl.multiple_of(s * ts, 8)
            pltpu.sync_copy(zeros_hbm.at[pl.ds(soff, ts)],
                            acc_sh.at[pl.ds(soff, ts)])
            pltpu.sync_copy(ones_hbm, vals_v)
            plsc.subcore_barrier()
            my_rounds = jnp.where(c == 0, jnp.int32(r0), jnp.int32(r1))
            cbase = jnp.where(c == 0, jnp.int32(0),
                              jnp.int32(r0)) * jnp.int32(_NS * _CH)
            tbase = cbase + s * jnp.int32(_CH)

            def body(i, carry):
                eoff = pl.multiple_of(tbase + i * jnp.int32(_NS * _CH), 8)
                pltpu.sync_copy(dst_hbm.at[pl.ds(eoff, _CH)], dst_v)
                pltpu.sync_copy(vals_v, acc_sh.at[dst_v], add=True)
                return carry

            lax.fori_loop(jnp.int32(0), my_rounds, body, jnp.int32(0))
            plsc.subcore_barrier()
            pltpu.sync_copy(acc_sh.at[pl.ds(soff, ts)],
                            out_hbm.at[c, pl.ds(soff, ts)])
        return k


def _tc_stage1(deg0, x2):
    rn = x2.shape[0]

    def body(deg_ref, x_ref, dinv_ref, u_ref, invdeg_ref):
        deg = deg_ref[0] + deg_ref[1] + 1.0
        dinv = lax.rsqrt(deg)
        dinv_ref[...] = dinv
        invdeg_ref[...] = 1.0 / deg
        u_ref[...] = dinv * x_ref[...]

    shp = jax.ShapeDtypeStruct((rn, _LANES), jnp.float32)
    return pl.pallas_call(body, out_shape=[shp, shp, shp])(deg0, x2)


def _tc_stage2(acc, dinv, x2, invdeg, aux):
    rn = x2.shape[0]

    def body(acc_ref, dinv_ref, x_ref, invdeg_ref, aux_ref, g_ref, v_ref):
        dinv = dinv_ref[...]
        s = dinv * (acc_ref[0] + acc_ref[1]) + x_ref[...] * invdeg_ref[...]
        g = jnp.zeros_like(s)
        for j in range(16):
            w1j = aux_ref[0, j]
            b1j = aux_ref[1, j]
            w2j = aux_ref[2, j]
            g = g + jnp.maximum(s * w1j + b1j, 0.0) * w2j
        g_ref[...] = g
        v_ref[...] = dinv * g

    shp = jax.ShapeDtypeStruct((rn, _LANES), jnp.float32)
    return pl.pallas_call(body, out_shape=[shp, shp])(
        acc, dinv, x2, invdeg, aux)


def _tc_stage3(acc, dinv, g, invdeg, aux):
    rn = g.shape[0]

    def body(acc_ref, dinv_ref, g_ref, invdeg_ref, aux_ref, out_ref):
        h = (dinv_ref[...] * (acc_ref[0] + acc_ref[1])
             + g_ref[...] * invdeg_ref[...] + aux_ref[3, 0])
        z = h - h
        lse = jnp.log(jnp.exp(z))
        out_ref[...] = z - lse

    shp = jax.ShapeDtypeStruct((rn, _LANES), jnp.float32)
    return pl.pallas_call(body, out_shape=shp)(acc, dinv, g, invdeg, aux)


def kernel(x, edge_index, W1, b1, W2, b2):
    n = x.shape[0]
    e = edge_index.shape[1]
    n_pad = _round_up(n + 1, _LANES * 8)
    rn = n_pad // _LANES
    e_pad = _round_up(e, _NS * _CH)
    dummy = n_pad - 1

    dst1 = edge_index[1].astype(jnp.int32)
    src1 = edge_index[0].astype(jnp.int32)
    if e_pad != e:
        pad = jnp.full((e_pad - e,), dummy, jnp.int32)
        dst1 = jnp.concatenate([dst1, pad])
        src1 = jnp.concatenate([src1, pad])

    x1 = jnp.zeros((n_pad,), jnp.float32).at[:n].set(x[:, 0].astype(jnp.float32))
    x2 = x1.reshape(rn, _LANES)
    zeros_n = jnp.zeros((n_pad,), jnp.float32)
    ones_ch = jnp.ones((_CH,), jnp.float32)

    aux = jnp.zeros((8, _LANES), jnp.float32)
    aux = aux.at[0, :16].set(W1[0].astype(jnp.float32))
    aux = aux.at[1, :16].set(b1.astype(jnp.float32))
    aux = aux.at[2, :16].set(W2[:, 0].astype(jnp.float32))
    aux = aux.at[3, 0].set(b2[0].astype(jnp.float32))

    count_k = _make_edge_pass(n_pad, e_pad, do_gather=False)
    gs_k = _make_edge_pass(n_pad, e_pad, do_gather=True)

    deg0 = count_k(dst1, ones_ch, zeros_n)
    dinv, u, invdeg = _tc_stage1(deg0.reshape(2, rn, _LANES), x2)

    acc1 = gs_k(src1, dst1, u.reshape(n_pad), zeros_n)
    g, v = _tc_stage2(acc1.reshape(2, rn, _LANES), dinv, x2, invdeg, aux)

    acc2 = gs_k(src1, dst1, v.reshape(n_pad), zeros_n)
    lsm = _tc_stage3(acc2.reshape(2, rn, _LANES), dinv, g, invdeg, aux)

    return lsm.reshape(n_pad)[:n].reshape(n, 1).astype(jnp.float64)


# speedup vs baseline: 2.0424x; 1.0879x over previous
import functools

import jax
import jax.numpy as jnp
from jax import lax
from jax.experimental import pallas as pl
from jax.experimental.pallas import tpu as pltpu
from jax.experimental.pallas import tpu_sc as plsc

_LANES = 128
_NS = 16
_CH = 10000
_R0_NUM = 1
_R0_DEN = 2


def _round_up(a, b):
    return (a + b - 1) // b * b


def _make_edge_pass(n_pad, e_pad, do_gather):
    rounds = e_pad // (_NS * _CH)
    r0 = rounds * _R0_NUM // _R0_DEN
    r1 = rounds - r0
    ts = n_pad // _NS
    mesh = plsc.VectorSubcoreMesh(core_axis_name="c", subcore_axis_name="s")

    out_type = jax.ShapeDtypeStruct((2, n_pad), jnp.float32)

    if do_gather:
        scratch = [
            pltpu.VMEM((_CH,), jnp.int32),
            pltpu.VMEM((_CH,), jnp.int32),
            pltpu.VMEM((_CH,), jnp.float32),
            pltpu.VMEM_SHARED((n_pad,), jnp.float32),
            pltpu.VMEM_SHARED((n_pad,), jnp.float32),
            pltpu.SemaphoreType.DMA,
        ]

        @functools.partial(pl.kernel, mesh=mesh, out_type=out_type,
                           scratch_types=scratch)
        def k(src_hbm, dst_hbm, table_hbm, zeros_hbm, out_hbm,
              src_v, dst_v, vals_v, acc_sh, tab_sh, sem):
            c = lax.axis_index("c").astype(jnp.int32)
            s = lax.axis_index("s").astype(jnp.int32)
            soff = pl.multiple_of(s * ts, 8)
            pltpu.sync_copy(table_hbm.at[pl.ds(soff, ts)],
                            tab_sh.at[pl.ds(soff, ts)])
            pltpu.sync_copy(zeros_hbm.at[pl.ds(soff, ts)],
                            acc_sh.at[pl.ds(soff, ts)])
            plsc.subcore_barrier()
            my_rounds = jnp.where(c == 0, jnp.int32(r0), jnp.int32(r1))
            cbase = jnp.where(c == 0, jnp.int32(0),
                              jnp.int32(r0)) * jnp.int32(_NS * _CH)
            tbase = cbase + s * jnp.int32(_CH)

            def body(i, carry):
                eoff = pl.multiple_of(tbase + i * jnp.int32(_NS * _CH), 8)
                pltpu.sync_copy(src_hbm.at[pl.ds(eoff, _CH)], src_v)
                pltpu.sync_copy(dst_hbm.at[pl.ds(eoff, _CH)], dst_v)
                pltpu.async_copy(tab_sh.at[src_v], vals_v, sem).wait()
                pltpu.sync_copy(vals_v, acc_sh.at[dst_v], add=True)
                return carry

            lax.fori_loop(jnp.int32(0), my_rounds, body, jnp.int32(0))
            plsc.subcore_barrier()
            pltpu.sync_copy(acc_sh.at[pl.ds(soff, ts)],
                            out_hbm.at[c, pl.ds(soff, ts)])
        return k
    else:
        scratch = [
            pltpu.VMEM((_CH,), jnp.int32),
            pltpu.VMEM((_CH,), jnp.float32),
            pltpu.VMEM_SHARED((n_pad,), jnp.float32),
            pltpu.SemaphoreType.DMA,
        ]

        @functools.partial(pl.kernel, mesh=mesh, out_type=out_type,
                           scratch_types=scratch)
        def k(dst_hbm, ones_hbm, zeros_hbm, out_hbm,
              dst_v, vals_v, acc_sh, sem):
            c = lax.axis_index("c").astype(jnp.int32)
            s = lax.axis_index("s").astype(jnp.int32)
            soff = pl.multiple_of(s * ts, 8)
            pltpu.sync_copy(zeros_hbm.at[pl.ds(soff, ts)],
                            acc_sh.at[pl.ds(soff, ts)])
            pltpu.sync_copy(ones_hbm, vals_v)
            plsc.subcore_barrier()
            my_rounds = jnp.where(c == 0, jnp.int32(r0), jnp.int32(r1))
            cbase = jnp.where(c == 0, jnp.int32(0),
                              jnp.int32(r0)) * jnp.int32(_NS * _CH)
            tbase = cbase + s * jnp.int32(_CH)

            def body(i, carry):
                eoff = pl.multiple_of(tbase + i * jnp.int32(_NS * _CH), 8)
                pltpu.sync_copy(dst_hbm.at[pl.ds(eoff, _CH)], dst_v)
                pltpu.sync_copy(vals_v, acc_sh.at[dst_v], add=True)
                return carry

            lax.fori_loop(jnp.int32(0), my_rounds, body, jnp.int32(0))
            plsc.subcore_barrier()
            pltpu.sync_copy(acc_sh.at[pl.ds(soff, ts)],
                            out_hbm.at[c, pl.ds(soff, ts)])
        return k


def _tc_stage1(deg0, x2):
    rn = x2.shape[0]

    def body(deg_ref, x_ref, dinv_ref, u_ref, invdeg_ref):
        deg = deg_ref[0] + deg_ref[1] + 1.0
        dinv = lax.rsqrt(deg)
        dinv_ref[...] = dinv
        invdeg_ref[...] = 1.0 / deg
        u_ref[...] = dinv * x_ref[...]

    shp = jax.ShapeDtypeStruct((rn, _LANES), jnp.float32)
    return pl.pallas_call(body, out_shape=[shp, shp, shp])(deg0, x2)


def _tc_stage2(acc, dinv, x2, invdeg, aux):
    rn = x2.shape[0]

    def body(acc_ref, dinv_ref, x_ref, invdeg_ref, aux_ref, g_ref, v_ref):
        dinv = dinv_ref[...]
        s = dinv * (acc_ref[0] + acc_ref[1]) + x_ref[...] * invdeg_ref[...]
        g = jnp.zeros_like(s)
        for j in range(16):
            w1j = aux_ref[0, j]
            b1j = aux_ref[1, j]
            w2j = aux_ref[2, j]
            g = g + jnp.maximum(s * w1j + b1j, 0.0) * w2j
        g_ref[...] = g
        v_ref[...] = dinv * g

    shp = jax.ShapeDtypeStruct((rn, _LANES), jnp.float32)
    return pl.pallas_call(body, out_shape=[shp, shp])(
        acc, dinv, x2, invdeg, aux)


def _tc_stage3(acc, dinv, g, invdeg, aux):
    rn = g.shape[0]

    def body(acc_ref, dinv_ref, g_ref, invdeg_ref, aux_ref, out_ref):
        h = (dinv_ref[...] * (acc_ref[0] + acc_ref[1])
             + g_ref[...] * invdeg_ref[...] + aux_ref[3, 0])
        z = h - h
        lse = jnp.log(jnp.exp(z))
        out_ref[...] = z - lse

    shp = jax.ShapeDtypeStruct((rn, _LANES), jnp.float32)
    return pl.pallas_call(body, out_shape=shp)(acc, dinv, g, invdeg, aux)


def kernel(x, edge_index, W1, b1, W2, b2):
    n = x.shape[0]
    e = edge_index.shape[1]
    n_pad = _round_up(n + 1, _LANES * 8)
    rn = n_pad // _LANES
    e_pad = _round_up(e, _NS * _CH)
    dummy = n_pad - 1

    ei32 = edge_index.astype(jnp.int32)
    src1 = ei32[0]
    dst1 = ei32[1]
    if e_pad != e:
        pad = jnp.full((e_pad - e,), dummy, jnp.int32)
        src1 = jnp.concatenate([src1, pad])
        dst1 = jnp.concatenate([dst1, pad])

    x1 = jnp.zeros((n_pad,), jnp.float32).at[:n].set(x[:, 0].astype(jnp.float32))
    x2 = x1.reshape(rn, _LANES)
    zeros_n = jnp.zeros((n_pad,), jnp.float32)
    ones_ch = jnp.ones((_CH,), jnp.float32)

    aux = jnp.zeros((8, _LANES), jnp.float32)
    aux = aux.at[0, :16].set(W1[0].astype(jnp.float32))
    aux = aux.at[1, :16].set(b1.astype(jnp.float32))
    aux = aux.at[2, :16].set(W2[:, 0].astype(jnp.float32))
    aux = aux.at[3, 0].set(b2[0].astype(jnp.float32))

    count_k = _make_edge_pass(n_pad, e_pad, do_gather=False)
    gs_k = _make_edge_pass(n_pad, e_pad, do_gather=True)

    deg0 = count_k(dst1, ones_ch, zeros_n)
    dinv, u, invdeg = _tc_stage1(deg0.reshape(2, rn, _LANES), x2)

    acc1 = gs_k(src1, dst1, u.reshape(n_pad), zeros_n)
    g, v = _tc_stage2(acc1.reshape(2, rn, _LANES), dinv, x2, invdeg, aux)

    acc2 = gs_k(src1, dst1, v.reshape(n_pad), zeros_n)
    lsm = _tc_stage3(acc2.reshape(2, rn, _LANES), dinv, g, invdeg, aux)

    return lsm.reshape(n_pad)[:n].reshape(n, 1).astype(jnp.float64)


# speedup vs baseline: 2.1670x; 1.0610x over previous
import functools

import jax
import jax.numpy as jnp
from jax import lax
from jax.experimental import pallas as pl
from jax.experimental.pallas import tpu as pltpu
from jax.experimental.pallas import tpu_sc as plsc

_LANES = 128
_NS = 16
_CH = 10000
_R0_NUM = 1
_R0_DEN = 2


def _round_up(a, b):
    return (a + b - 1) // b * b


def _make_edge_pass(n_pad, e_pad, do_gather):
    rounds = e_pad // (_NS * _CH)
    r0 = rounds * _R0_NUM // _R0_DEN
    r1 = rounds - r0
    ts = n_pad // _NS
    mesh = plsc.VectorSubcoreMesh(core_axis_name="c", subcore_axis_name="s")

    out_type = jax.ShapeDtypeStruct((2, n_pad), jnp.float32)

    if do_gather:
        scratch = [
            pltpu.VMEM((_CH,), jnp.int32),
            pltpu.VMEM((_CH,), jnp.int32),
            pltpu.VMEM((_CH,), jnp.int32),
            pltpu.VMEM((_CH,), jnp.int32),
            pltpu.VMEM((_CH,), jnp.float32),
            pltpu.VMEM_SHARED((n_pad,), jnp.float32),
            pltpu.VMEM_SHARED((n_pad,), jnp.float32),
            pltpu.SemaphoreType.DMA,
            pltpu.SemaphoreType.DMA,
        ]

        @functools.partial(pl.kernel, mesh=mesh, out_type=out_type,
                           scratch_types=scratch)
        def k(src_hbm, dst_hbm, table_hbm, zeros_hbm, out_hbm,
              src_a, dst_a, src_b, dst_b, vals_v, acc_sh, tab_sh,
              sem, psem):
            c = lax.axis_index("c").astype(jnp.int32)
            s = lax.axis_index("s").astype(jnp.int32)
            soff = pl.multiple_of(s * ts, 8)
            pltpu.sync_copy(table_hbm.at[pl.ds(soff, ts)],
                            tab_sh.at[pl.ds(soff, ts)])
            pltpu.sync_copy(zeros_hbm.at[pl.ds(soff, ts)],
                            acc_sh.at[pl.ds(soff, ts)])
            plsc.subcore_barrier()
            my_rounds = jnp.where(c == 0, jnp.int32(r0), jnp.int32(r1))
            cbase = jnp.where(c == 0, jnp.int32(0),
                              jnp.int32(r0)) * jnp.int32(_NS * _CH)
            tbase = cbase + s * jnp.int32(_CH)

            def eoff(i):
                return pl.multiple_of(tbase + i * jnp.int32(_NS * _CH), 8)

            def start_idx(i, sv, dv):
                pltpu.make_async_copy(src_hbm.at[pl.ds(eoff(i), _CH)],
                                      sv, psem).start()
                pltpu.make_async_copy(dst_hbm.at[pl.ds(eoff(i), _CH)],
                                      dv, psem).start()

            def wait_idx(sv, dv):
                pltpu.make_async_copy(src_hbm.at[pl.ds(eoff(jnp.int32(0)),
                                                       _CH)], sv, psem).wait()
                pltpu.make_async_copy(dst_hbm.at[pl.ds(eoff(jnp.int32(0)),
                                                       _CH)], dv, psem).wait()

            def process(sv, dv):
                pltpu.async_copy(tab_sh.at[sv], vals_v, sem).wait()
                pltpu.sync_copy(vals_v, acc_sh.at[dv], add=True)

            @pl.when(my_rounds >= jnp.int32(2))
            def _():
                start_idx(jnp.int32(0), src_a, dst_a)

            def pair_body(i2, carry):
                ra = i2 * jnp.int32(2)
                wait_idx(src_a, dst_a)
                start_idx(ra + jnp.int32(1), src_b, dst_b)
                process(src_a, dst_a)
                wait_idx(src_b, dst_b)

                paired = (my_rounds // jnp.int32(2)) * jnp.int32(2)

                @pl.when(ra + jnp.int32(2) < paired)
                def _():
                    start_idx(ra + jnp.int32(2), src_a, dst_a)

                process(src_b, dst_b)
                return carry

            lax.fori_loop(jnp.int32(0), my_rounds // jnp.int32(2), pair_body,
                          jnp.int32(0))

            @pl.when(my_rounds % jnp.int32(2) == jnp.int32(1))
            def _():
                i = my_rounds - jnp.int32(1)
                pltpu.sync_copy(src_hbm.at[pl.ds(eoff(i), _CH)], src_a)
                pltpu.sync_copy(dst_hbm.at[pl.ds(eoff(i), _CH)], dst_a)
                process(src_a, dst_a)

            plsc.subcore_barrier()
            pltpu.sync_copy(acc_sh.at[pl.ds(soff, ts)],
                            out_hbm.at[c, pl.ds(soff, ts)])
        return k
    else:
        scratch = [
            pltpu.VMEM((_CH,), jnp.int32),
            pltpu.VMEM((_CH,), jnp.float32),
            pltpu.VMEM_SHARED((n_pad,), jnp.float32),
            pltpu.SemaphoreType.DMA,
        ]

        @functools.partial(pl.kernel, mesh=mesh, out_type=out_type,
                           scratch_types=scratch)
        def k(dst_hbm, ones_hbm, zeros_hbm, out_hbm,
              dst_v, vals_v, acc_sh, sem):
            c = lax.axis_index("c").astype(jnp.int32)
            s = lax.axis_index("s").astype(jnp.int32)
            soff = pl.multiple_of(s * ts, 8)
            pltpu.sync_copy(zeros_hbm.at[pl.ds(soff, ts)],
                            acc_sh.at[pl.ds(soff, ts)])
            pltpu.sync_copy(ones_hbm, vals_v)
            plsc.subcore_barrier()
            my_rounds = jnp.where(c == 0, jnp.int32(r0), jnp.int32(r1))
            cbase = jnp.where(c == 0, jnp.int32(0),
                              jnp.int32(r0)) * jnp.int32(_NS * _CH)
            tbase = cbase + s * jnp.int32(_CH)

            def body(i, carry):
                eoff = pl.multiple_of(tbase + i * jnp.int32(_NS * _CH), 8)
                pltpu.sync_copy(dst_hbm.at[pl.ds(eoff, _CH)], dst_v)
                pltpu.sync_copy(vals_v, acc_sh.at[dst_v], add=True)
                return carry

            lax.fori_loop(jnp.int32(0), my_rounds, body, jnp.int32(0))
            plsc.subcore_barrier()
            pltpu.sync_copy(acc_sh.at[pl.ds(soff, ts)],
                            out_hbm.at[c, pl.ds(soff, ts)])
        return k


def _tc_stage1(deg0, x2):
    rn = x2.shape[0]

    def body(deg_ref, x_ref, dinv_ref, u_ref, invdeg_ref):
        deg = deg_ref[0] + deg_ref[1] + 1.0
        dinv = lax.rsqrt(deg)
        dinv_ref[...] = dinv
        invdeg_ref[...] = 1.0 / deg
        u_ref[...] = dinv * x_ref[...]

    shp = jax.ShapeDtypeStruct((rn, _LANES), jnp.float32)
    return pl.pallas_call(body, out_shape=[shp, shp, shp])(deg0, x2)


def _tc_stage2(acc, dinv, x2, invdeg, aux):
    rn = x2.shape[0]

    def body(acc_ref, dinv_ref, x_ref, invdeg_ref, aux_ref, g_ref, v_ref):
        dinv = dinv_ref[...]
        s = dinv * (acc_ref[0] + acc_ref[1]) + x_ref[...] * invdeg_ref[...]
        g = jnp.zeros_like(s)
        for j in range(16):
            w1j = aux_ref[0, j]
            b1j = aux_ref[1, j]
            w2j = aux_ref[2, j]
            g = g + jnp.maximum(s * w1j + b1j, 0.0) * w2j
        g_ref[...] = g
        v_ref[...] = dinv * g

    shp = jax.ShapeDtypeStruct((rn, _LANES), jnp.float32)
    return pl.pallas_call(body, out_shape=[shp, shp])(
        acc, dinv, x2, invdeg, aux)


def _tc_stage3(acc, dinv, g, invdeg, aux):
    rn = g.shape[0]

    def body(acc_ref, dinv_ref, g_ref, invdeg_ref, aux_ref, out_ref):
        h = (dinv_ref[...] * (acc_ref[0] + acc_ref[1])
             + g_ref[...] * invdeg_ref[...] + aux_ref[3, 0])
        z = h - h
        lse = jnp.log(jnp.exp(z))
        out_ref[...] = z - lse

    shp = jax.ShapeDtypeStruct((rn, _LANES), jnp.float32)
    return pl.pallas_call(body, out_shape=shp)(acc, dinv, g, invdeg, aux)


def kernel(x, edge_index, W1, b1, W2, b2):
    n = x.shape[0]
    e = edge_index.shape[1]
    n_pad = _round_up(n + 1, _LANES * 8)
    rn = n_pad // _LANES
    e_pad = _round_up(e, _NS * _CH)
    dummy = n_pad - 1

    ei32 = edge_index.astype(jnp.int32)
    src1 = ei32[0]
    dst1 = ei32[1]
    if e_pad != e:
        pad = jnp.full((e_pad - e,), dummy, jnp.int32)
        src1 = jnp.concatenate([src1, pad])
        dst1 = jnp.concatenate([dst1, pad])

    x1 = jnp.zeros((n_pad,), jnp.float32).at[:n].set(x[:, 0].astype(jnp.float32))
    x2 = x1.reshape(rn, _LANES)
    zeros_n = jnp.zeros((n_pad,), jnp.float32)
    ones_ch = jnp.ones((_CH,), jnp.float32)

    aux = jnp.zeros((8, _LANES), jnp.float32)
    aux = aux.at[0, :16].set(W1[0].astype(jnp.float32))
    aux = aux.at[1, :16].set(b1.astype(jnp.float32))
    aux = aux.at[2, :16].set(W2[:, 0].astype(jnp.float32))
    aux = aux.at[3, 0].set(b2[0].astype(jnp.float32))

    count_k = _make_edge_pass(n_pad, e_pad, do_gather=False)
    gs_k = _make_edge_pass(n_pad, e_pad, do_gather=True)

    deg0 = count_k(dst1, ones_ch, zeros_n)
    dinv, u, invdeg = _tc_stage1(deg0.reshape(2, rn, _LANES), x2)

    acc1 = gs_k(src1, dst1, u.reshape(n_pad), zeros_n)
    g, v = _tc_stage2(acc1.reshape(2, rn, _LANES), dinv, x2, invdeg, aux)

    acc2 = gs_k(src1, dst1, v.reshape(n_pad), zeros_n)
    lsm = _tc_stage3(acc2.reshape(2, rn, _LANES), dinv, g, invdeg, aux)

    return lsm.reshape(n_pad)[:n].reshape(n, 1).astype(jnp.float64)


# speedup vs baseline: 2.2074x; 1.0186x over previous
import functools

import jax
import jax.numpy as jnp
from jax import lax
from jax.experimental import pallas as pl
from jax.experimental.pallas import tpu as pltpu
from jax.experimental.pallas import tpu_sc as plsc

_LANES = 128
_NS = 16
_CH = 10000
_R0_NUM = 1
_R0_DEN = 2


def _round_up(a, b):
    return (a + b - 1) // b * b


def _make_edge_pass(n_pad, e_pad, do_gather):
    rounds = e_pad // (_NS * _CH)
    r0 = rounds * _R0_NUM // _R0_DEN
    r1 = rounds - r0
    ts = n_pad // _NS
    mesh = plsc.VectorSubcoreMesh(core_axis_name="c", subcore_axis_name="s")

    out_type = jax.ShapeDtypeStruct((2, n_pad), jnp.float32)

    if do_gather:
        scratch = [
            pltpu.VMEM((_CH,), jnp.int32),
            pltpu.VMEM((_CH,), jnp.int32),
            pltpu.VMEM((_CH,), jnp.int32),
            pltpu.VMEM((_CH,), jnp.int32),
            pltpu.VMEM((_CH,), jnp.float32),
            pltpu.VMEM_SHARED((n_pad,), jnp.float32),
            pltpu.VMEM_SHARED((n_pad,), jnp.float32),
            pltpu.SemaphoreType.DMA,
            pltpu.SemaphoreType.DMA,
        ]

        @functools.partial(pl.kernel, mesh=mesh, out_type=out_type,
                           scratch_types=scratch)
        def k(src_hbm, dst_hbm, table_hbm, zeros_hbm, out_hbm,
              src_a, dst_a, src_b, dst_b, vals_v, acc_sh, tab_sh,
              sem, psem):
            c = lax.axis_index("c").astype(jnp.int32)
            s = lax.axis_index("s").astype(jnp.int32)
            soff = pl.multiple_of(s * ts, 8)
            pltpu.sync_copy(table_hbm.at[pl.ds(soff, ts)],
                            tab_sh.at[pl.ds(soff, ts)])
            pltpu.sync_copy(zeros_hbm.at[pl.ds(soff, ts)],
                            acc_sh.at[pl.ds(soff, ts)])
            plsc.subcore_barrier()
            my_rounds = jnp.where(c == 0, jnp.int32(r0), jnp.int32(r1))
            cbase = jnp.where(c == 0, jnp.int32(0),
                              jnp.int32(r0)) * jnp.int32(_NS * _CH)
            tbase = cbase + s * jnp.int32(_CH)

            def eoff(i):
                return pl.multiple_of(tbase + i * jnp.int32(_NS * _CH), 8)

            def start_idx(i, sv, dv):
                pltpu.make_async_copy(src_hbm.at[pl.ds(eoff(i), _CH)],
                                      sv, psem).start()
                pltpu.make_async_copy(dst_hbm.at[pl.ds(eoff(i), _CH)],
                                      dv, psem).start()

            def wait_idx(sv, dv):
                pltpu.make_async_copy(src_hbm.at[pl.ds(eoff(jnp.int32(0)),
                                                       _CH)], sv, psem).wait()
                pltpu.make_async_copy(dst_hbm.at[pl.ds(eoff(jnp.int32(0)),
                                                       _CH)], dv, psem).wait()

            def process(sv, dv):
                pltpu.async_copy(tab_sh.at[sv], vals_v, sem).wait()
                pltpu.sync_copy(vals_v, acc_sh.at[dv], add=True)

            @pl.when(my_rounds >= jnp.int32(2))
            def _():
                start_idx(jnp.int32(0), src_a, dst_a)

            def pair_body(i2, carry):
                ra = i2 * jnp.int32(2)
                wait_idx(src_a, dst_a)
                start_idx(ra + jnp.int32(1), src_b, dst_b)
                process(src_a, dst_a)
                wait_idx(src_b, dst_b)

                paired = (my_rounds // jnp.int32(2)) * jnp.int32(2)

                @pl.when(ra + jnp.int32(2) < paired)
                def _():
                    start_idx(ra + jnp.int32(2), src_a, dst_a)

                process(src_b, dst_b)
                return carry

            lax.fori_loop(jnp.int32(0), my_rounds // jnp.int32(2), pair_body,
                          jnp.int32(0))

            @pl.when(my_rounds % jnp.int32(2) == jnp.int32(1))
            def _():
                i = my_rounds - jnp.int32(1)
                pltpu.sync_copy(src_hbm.at[pl.ds(eoff(i), _CH)], src_a)
                pltpu.sync_copy(dst_hbm.at[pl.ds(eoff(i), _CH)], dst_a)
                process(src_a, dst_a)

            plsc.subcore_barrier()
            pltpu.sync_copy(acc_sh.at[pl.ds(soff, ts)],
                            out_hbm.at[c, pl.ds(soff, ts)])
        return k
    else:
        scratch = [
            pltpu.VMEM((_CH,), jnp.int32),
            pltpu.VMEM((_CH,), jnp.int32),
            pltpu.VMEM((_CH,), jnp.float32),
            pltpu.VMEM_SHARED((n_pad,), jnp.float32),
            pltpu.SemaphoreType.DMA,
        ]

        @functools.partial(pl.kernel, mesh=mesh, out_type=out_type,
                           scratch_types=scratch)
        def k(dst_hbm, ones_hbm, zeros_hbm, out_hbm,
              dst_a, dst_b, vals_v, acc_sh, psem):
            c = lax.axis_index("c").astype(jnp.int32)
            s = lax.axis_index("s").astype(jnp.int32)
            soff = pl.multiple_of(s * ts, 8)
            pltpu.sync_copy(zeros_hbm.at[pl.ds(soff, ts)],
                            acc_sh.at[pl.ds(soff, ts)])
            pltpu.sync_copy(ones_hbm, vals_v)
            plsc.subcore_barrier()
            my_rounds = jnp.where(c == 0, jnp.int32(r0), jnp.int32(r1))
            cbase = jnp.where(c == 0, jnp.int32(0),
                              jnp.int32(r0)) * jnp.int32(_NS * _CH)
            tbase = cbase + s * jnp.int32(_CH)

            def eoff(i):
                return pl.multiple_of(tbase + i * jnp.int32(_NS * _CH), 8)

            def start_idx(i, dv):
                pltpu.make_async_copy(dst_hbm.at[pl.ds(eoff(i), _CH)],
                                      dv, psem).start()

            def wait_idx(dv):
                pltpu.make_async_copy(dst_hbm.at[pl.ds(eoff(jnp.int32(0)),
                                                       _CH)], dv, psem).wait()

            def process(dv):
                pltpu.sync_copy(vals_v, acc_sh.at[dv], add=True)

            @pl.when(my_rounds >= jnp.int32(2))
            def _():
                start_idx(jnp.int32(0), dst_a)

            def pair_body(i2, carry):
                ra = i2 * jnp.int32(2)
                wait_idx(dst_a)
                start_idx(ra + jnp.int32(1), dst_b)
                process(dst_a)
                wait_idx(dst_b)
                paired = (my_rounds // jnp.int32(2)) * jnp.int32(2)

                @pl.when(ra + jnp.int32(2) < paired)
                def _():
                    start_idx(ra + jnp.int32(2), dst_a)

                process(dst_b)
                return carry

            lax.fori_loop(jnp.int32(0), my_rounds // jnp.int32(2), pair_body,
                          jnp.int32(0))

            @pl.when(my_rounds % jnp.int32(2) == jnp.int32(1))
            def _():
                i = my_rounds - jnp.int32(1)
                pltpu.sync_copy(dst_hbm.at[pl.ds(eoff(i), _CH)], dst_a)
                process(dst_a)

            plsc.subcore_barrier()
            pltpu.sync_copy(acc_sh.at[pl.ds(soff, ts)],
                            out_hbm.at[c, pl.ds(soff, ts)])
        return k


def _tc_stage1(deg0, x2):
    rn = x2.shape[0]

    def body(deg_ref, x_ref, dinv_ref, u_ref, invdeg_ref):
        deg = deg_ref[0] + deg_ref[1] + 1.0
        dinv = lax.rsqrt(deg)
        dinv_ref[...] = dinv
        invdeg_ref[...] = 1.0 / deg
        u_ref[...] = dinv * x_ref[...]

    shp = jax.ShapeDtypeStruct((rn, _LANES), jnp.float32)
    return pl.pallas_call(body, out_shape=[shp, shp, shp])(deg0, x2)


def _tc_stage2(acc, dinv, x2, invdeg, aux):
    rn = x2.shape[0]

    def body(acc_ref, dinv_ref, x_ref, invdeg_ref, aux_ref, g_ref, v_ref):
        dinv = dinv_ref[...]
        s = dinv * (acc_ref[0] + acc_ref[1]) + x_ref[...] * invdeg_ref[...]
        g = jnp.zeros_like(s)
        for j in range(16):
            w1j = aux_ref[0, j]
            b1j = aux_ref[1, j]
            w2j = aux_ref[2, j]
            g = g + jnp.maximum(s * w1j + b1j, 0.0) * w2j
        g_ref[...] = g
        v_ref[...] = dinv * g

    shp = jax.ShapeDtypeStruct((rn, _LANES), jnp.float32)
    return pl.pallas_call(body, out_shape=[shp, shp])(
        acc, dinv, x2, invdeg, aux)


def _tc_stage3(acc, dinv, g, invdeg, aux):
    rn = g.shape[0]

    def body(acc_ref, dinv_ref, g_ref, invdeg_ref, aux_ref, out_ref):
        h = (dinv_ref[...] * (acc_ref[0] + acc_ref[1])
             + g_ref[...] * invdeg_ref[...] + aux_ref[3, 0])
        z = h - h
        lse = jnp.log(jnp.exp(z))
        out_ref[...] = z - lse

    shp = jax.ShapeDtypeStruct((rn, _LANES), jnp.float32)
    return pl.pallas_call(body, out_shape=shp)(acc, dinv, g, invdeg, aux)


def kernel(x, edge_index, W1, b1, W2, b2):
    n = x.shape[0]
    e = edge_index.shape[1]
    n_pad = _round_up(n + 1, _LANES * 8)
    rn = n_pad // _LANES
    e_pad = _round_up(e, _NS * _CH)
    dummy = n_pad - 1

    ei32 = edge_index.astype(jnp.int32)
    src1 = ei32[0]
    dst1 = ei32[1]
    if e_pad != e:
        pad = jnp.full((e_pad - e,), dummy, jnp.int32)
        src1 = jnp.concatenate([src1, pad])
        dst1 = jnp.concatenate([dst1, pad])

    x1 = jnp.zeros((n_pad,), jnp.float32).at[:n].set(x[:, 0].astype(jnp.float32))
    x2 = x1.reshape(rn, _LANES)
    zeros_n = jnp.zeros((n_pad,), jnp.float32)
    ones_ch = jnp.ones((_CH,), jnp.float32)

    aux = jnp.zeros((8, _LANES), jnp.float32)
    aux = aux.at[0, :16].set(W1[0].astype(jnp.float32))
    aux = aux.at[1, :16].set(b1.astype(jnp.float32))
    aux = aux.at[2, :16].set(W2[:, 0].astype(jnp.float32))
    aux = aux.at[3, 0].set(b2[0].astype(jnp.float32))

    count_k = _make_edge_pass(n_pad, e_pad, do_gather=False)
    gs_k = _make_edge_pass(n_pad, e_pad, do_gather=True)

    deg0 = count_k(dst1, ones_ch, zeros_n)
    dinv, u, invdeg = _tc_stage1(deg0.reshape(2, rn, _LANES), x2)

    acc1 = gs_k(src1, dst1, u.reshape(n_pad), zeros_n)
    g, v = _tc_stage2(acc1.reshape(2, rn, _LANES), dinv, x2, invdeg, aux)

    acc2 = gs_k(src1, dst1, v.reshape(n_pad), zeros_n)
    lsm = _tc_stage3(acc2.reshape(2, rn, _LANES), dinv, g, invdeg, aux)

    return lsm.reshape(n_pad)[:n].reshape(n, 1).astype(jnp.float64)
